# baseline (speedup 1.0000x reference)
import functools

import jax
import jax.numpy as jnp
from jax import lax
from jax.experimental import pallas as pl
from jax.experimental.pallas import tpu as pltpu
from jax.experimental.pallas import tpu_sc as plsc

D_MODEL = 1024
VOCAB = 32000
BATCH = 4
SEQ = 2048
IMG_LEN = 256

NTOK = BATCH * SEQ
NC, NS = 2, 16
NW = NC * NS
PER_W = NTOK // NW
CHUNK = 64
NCHUNK = PER_W // CHUNK

OUT_ROWS = BATCH * (IMG_LEN + SEQ)
BLK = 256
BPB = (IMG_LEN + SEQ) // BLK
GRID = OUT_ROWS // BLK
EMB_BLOCKS = NTOK // BLK


@functools.lru_cache(maxsize=None)
def _build_gather():
    mesh = plsc.VectorSubcoreMesh(core_axis_name="c", subcore_axis_name="s")

    @functools.partial(
        pl.kernel,
        mesh=mesh,
        out_type=jax.ShapeDtypeStruct((NTOK, D_MODEL), jnp.float32),
        scratch_types=[
            pltpu.VMEM((CHUNK,), jnp.int32),
            pltpu.VMEM((CHUNK, D_MODEL), jnp.float32),
            pltpu.SemaphoreType.DMA,
        ],
    )
    def _gather(ids_hbm, table_hbm, out_hbm, idx_v, rows_v, sem):
        wid = lax.axis_index("s") * NC + lax.axis_index("c")
        base = wid * PER_W
        for c in range(NCHUNK):
            off = base + c * CHUNK
            pltpu.sync_copy(ids_hbm.at[pl.ds(off, CHUNK)], idx_v)
            pltpu.async_copy(table_hbm.at[idx_v], rows_v, sem).wait()
            pltpu.sync_copy(rows_v, out_hbm.at[pl.ds(off, CHUNK)])

    return _gather


def _mm_body(img_ref, emb_ref, w_ref, b_ref, out_ref):
    jb = pl.program_id(0) % BPB

    @pl.when(jb == 0)
    def _():
        x = img_ref[...].astype(jnp.bfloat16)
        out_ref[...] = (
            jnp.dot(x, w_ref[...], preferred_element_type=jnp.float32)
            + b_ref[...]
        )

    @pl.when(jb != 0)
    def _():
        x = emb_ref[...].astype(jnp.bfloat16)
        out_ref[...] = (
            jnp.dot(x, w_ref[...], preferred_element_type=jnp.float32)
            + b_ref[...]
        )


@functools.lru_cache(maxsize=None)
def _build_matmul():
    return pl.pallas_call(
        _mm_body,
        grid=(GRID,),
        in_specs=[
            pl.BlockSpec((BLK, D_MODEL), lambda j: (j // BPB, 0)),
            pl.BlockSpec(
                (BLK, D_MODEL),
                lambda j: (jnp.clip(j - j // BPB - 1, 0, EMB_BLOCKS - 1), 0),
            ),
            pl.BlockSpec((D_MODEL, D_MODEL), lambda j: (0, 0)),
            pl.BlockSpec((1, D_MODEL), lambda j: (0, 0)),
        ],
        out_specs=pl.BlockSpec((BLK, D_MODEL), lambda j: (j, 0)),
        out_shape=jax.ShapeDtypeStruct((OUT_ROWS, D_MODEL), jnp.float32),
        compiler_params=pltpu.CompilerParams(
            dimension_semantics=("arbitrary",),
        ),
    )


def kernel(input_ids, image_features, table, W, b):
    ids_flat = input_ids.reshape(NTOK)
    emb = _build_gather()(ids_flat, table)
    img2d = image_features.reshape(BATCH * IMG_LEN, D_MODEL)
    out2d = _build_matmul()(img2d, emb, W.astype(jnp.bfloat16), b.reshape(1, D_MODEL))
    return out2d.reshape(BATCH, IMG_LEN + SEQ, D_MODEL)


# speedup vs baseline: 1.0960x; 1.0960x over previous
import functools

import jax
import jax.numpy as jnp
from jax import lax
from jax.experimental import pallas as pl
from jax.experimental.pallas import tpu as pltpu
from jax.experimental.pallas import tpu_sc as plsc

D_MODEL = 1024
VOCAB = 32000
BATCH = 4
SEQ = 2048
IMG_LEN = 256

NTOK = BATCH * SEQ
HALF = NTOK // 2
NC, NS = 2, 16
NW = NC * NS
PER_W = HALF // NW
CHUNK = 64
NCHUNK = PER_W // CHUNK

OUT_ROWS = BATCH * (IMG_LEN + SEQ)
BLK = 256
BPB = (IMG_LEN + SEQ) // BLK
IMG_BLOCKS = BATCH * IMG_LEN // BLK
EMB_BLOCKS_H = HALF // BLK


@functools.lru_cache(maxsize=None)
def _build_gather():
    mesh = plsc.VectorSubcoreMesh(core_axis_name="c", subcore_axis_name="s")

    @functools.partial(
        pl.kernel,
        mesh=mesh,
        out_type=jax.ShapeDtypeStruct((HALF, D_MODEL), jnp.float32),
        scratch_types=[
            pltpu.VMEM((CHUNK,), jnp.int32),
            pltpu.VMEM((CHUNK, D_MODEL), jnp.float32),
            pltpu.SemaphoreType.DMA,
        ],
    )
    def _gather(ids_hbm, table_hbm, out_hbm, idx_v, rows_v, sem):
        wid = lax.axis_index("s") * NC + lax.axis_index("c")
        base = wid * PER_W
        for c in range(NCHUNK):
            off = base + c * CHUNK
            pltpu.sync_copy(ids_hbm.at[pl.ds(off, CHUNK)], idx_v)
            pltpu.async_copy(table_hbm.at[idx_v], rows_v, sem).wait()
            pltpu.sync_copy(rows_v, out_hbm.at[pl.ds(off, CHUNK)])

    return _gather


def _mm_img_body(img_ref, w_ref, b_ref, out_ref):
    x = img_ref[...].astype(jnp.bfloat16)
    out_ref[...] = (
        jnp.dot(x, w_ref[...], preferred_element_type=jnp.float32) + b_ref[...]
    )


def _mm_emb_body(prev_ref, emb_ref, w_ref, b_ref, out_ref):
    del prev_ref
    x = emb_ref[...].astype(jnp.bfloat16)
    out_ref[...] = (
        jnp.dot(x, w_ref[...], preferred_element_type=jnp.float32) + b_ref[...]
    )


@functools.lru_cache(maxsize=None)
def _build_mm_img():
    return pl.pallas_call(
        _mm_img_body,
        grid=(IMG_BLOCKS,),
        in_specs=[
            pl.BlockSpec((BLK, D_MODEL), lambda j: (j, 0)),
            pl.BlockSpec((D_MODEL, D_MODEL), lambda j: (0, 0)),
            pl.BlockSpec((1, D_MODEL), lambda j: (0, 0)),
        ],
        out_specs=pl.BlockSpec((BLK, D_MODEL), lambda j: (j * BPB, 0)),
        out_shape=jax.ShapeDtypeStruct((OUT_ROWS, D_MODEL), jnp.float32),
        compiler_params=pltpu.CompilerParams(
            dimension_semantics=("arbitrary",),
        ),
    )


@functools.lru_cache(maxsize=None)
def _build_mm_emb(half: int):
    def out_map(j, half=half):
        return ((half * 2 + j // 8) * BPB + 1 + j % 8, 0)

    return pl.pallas_call(
        _mm_emb_body,
        grid=(EMB_BLOCKS_H,),
        in_specs=[
            pl.BlockSpec(memory_space=pl.ANY),
            pl.BlockSpec((BLK, D_MODEL), lambda j: (j, 0)),
            pl.BlockSpec((D_MODEL, D_MODEL), lambda j: (0, 0)),
            pl.BlockSpec((1, D_MODEL), lambda j: (0, 0)),
        ],
        out_specs=pl.BlockSpec((BLK, D_MODEL), out_map),
        out_shape=jax.ShapeDtypeStruct((OUT_ROWS, D_MODEL), jnp.float32),
        input_output_aliases={0: 0},
        compiler_params=pltpu.CompilerParams(
            dimension_semantics=("arbitrary",),
        ),
    )


def kernel(input_ids, image_features, table, W, b):
    ids_flat = input_ids.reshape(NTOK)
    gather = _build_gather()
    emb_a = gather(ids_flat[:HALF], table)
    emb_b = gather(ids_flat[HALF:], table)
    img2d = image_features.reshape(BATCH * IMG_LEN, D_MODEL)
    w_bf = W.astype(jnp.bfloat16)
    b2d = b.reshape(1, D_MODEL)
    out = _build_mm_img()(img2d, w_bf, b2d)
    out = _build_mm_emb(0)(out, emb_a, w_bf, b2d)
    out = _build_mm_emb(1)(out, emb_b, w_bf, b2d)
    return out.reshape(BATCH, IMG_LEN + SEQ, D_MODEL)
